# baseline (device time: 53197 ns/iter reference)
import jax
import jax.numpy as jnp
from jax import lax
from jax.experimental import pallas as pl
from jax.experimental.pallas import tpu as pltpu

N_DEV = 4
SQ = 1024
SKV = 1024
D_MODEL = 1024
H_PER = 8
DH = 128
BLK = 64
SCALE = 0.08838834764831843
BF = jnp.bfloat16


def kernel(x, Wq, K_ext, V_ext, Wo):
    x2 = x[0].astype(BF)
    wq2 = Wq.astype(BF)
    wo2 = Wo.astype(BF)

    def body(x_ref, wq_ref, kext_ref, vext_ref, wo_ref, out_ref,
             ctx_ref, pbuf, stag_cw, stag_ccw, kf32, vf32, kbf, vbf,
             send_cw, recv_cw, send_ccw, recv_ccw, kv_sems):
        pos = lax.axis_index("i")
        left = (pos - 1) % N_DEV
        right = (pos + 1) % N_DEV

        kdma = pltpu.make_async_copy(
            kext_ref.at[0, :, pl.ds(pos * H_PER, H_PER), :], kf32,
            kv_sems.at[0])
        vdma = pltpu.make_async_copy(
            vext_ref.at[0, :, pl.ds(pos * H_PER, H_PER), :], vf32,
            kv_sems.at[1])
        kdma.start()
        vdma.start()

        barrier_sem = pltpu.get_barrier_semaphore()
        for nbr in [left, right]:
            pl.semaphore_signal(
                barrier_sem, inc=1,
                device_id=(nbr,), device_id_type=pl.DeviceIdType.MESH,
            )

        k_ref = kbf
        v_ref = vbf

        RCH = SQ // N_DEV

        def rows(c):
            return pl.ds(c * RCH, RCH)

        CL = pl.ds(0, D_MODEL // 2)
        CR = pl.ds(D_MODEL // 2, D_MODEL // 2)

        def compute_chunk(c, q=None):
            if q is None:
                q = jnp.dot(x_ref[rows(c), :], wq_ref[:, :],
                            preferred_element_type=jnp.float32)
            qb = (lax.broadcasted_iota(jnp.int32, (RCH, SKV), 0)
                  + c * RCH) // BLK
            kb = lax.broadcasted_iota(jnp.int32, (RCH, SKV), 1) // BLK
            mask = (qb == kb) | (kb == 0) | ((qb + kb) % 3 == 0)
            maskf = mask.astype(jnp.float32)
            for h in range(H_PER):
                sl = pl.ds(h * DH, DH)
                qh = q[:, h * DH:(h + 1) * DH].astype(BF)
                kh = k_ref[:, sl]
                scores = lax.dot_general(
                    qh, kh, (((1,), (1,)), ((), ())),
                    preferred_element_type=jnp.float32,
                )
                w = jnp.exp(scores * SCALE) * maskf
                wsum = jnp.sum(w, axis=-1, keepdims=True)
                ctx = jnp.dot(w.astype(BF), v_ref[:, sl],
                              preferred_element_type=jnp.float32)
                ctx_ref[:, sl] = (ctx / wsum).astype(BF)
            pbuf[rows(c), :] = jnp.dot(
                ctx_ref[:, :], wo_ref[:, :],
                preferred_element_type=jnp.float32).astype(BF)

        def cw_rdma(s):
            return pltpu.make_async_remote_copy(
                src_ref=pbuf.at[rows((pos - s) % N_DEV), CL],
                dst_ref=stag_cw.at[s],
                send_sem=send_cw.at[s], recv_sem=recv_cw.at[s],
                device_id=(right,), device_id_type=pl.DeviceIdType.MESH,
            )

        def ccw_rdma(s):
            return pltpu.make_async_remote_copy(
                src_ref=pbuf.at[rows((pos + s) % N_DEV), CR],
                dst_ref=stag_ccw.at[s],
                send_sem=send_ccw.at[s], recv_sem=recv_ccw.at[s],
                device_id=(left,), device_id_type=pl.DeviceIdType.MESH,
            )

        def accum(c, half, stag, s):
            cur = pbuf[rows(c), half].astype(jnp.float32)
            inc = stag[s].astype(jnp.float32)
            pbuf[rows(c), half] = (cur + inc).astype(BF)

        def ag_cw(d):
            return pltpu.make_async_remote_copy(
                src_ref=pbuf.at[rows((pos + 1) % N_DEV), CL],
                dst_ref=pbuf.at[rows((pos + 1) % N_DEV), CL],
                send_sem=send_cw.at[2 + d], recv_sem=recv_cw.at[2 + d],
                device_id=((pos + d) % N_DEV,),
                device_id_type=pl.DeviceIdType.MESH,
            )

        def ag_ccw(d):
            return pltpu.make_async_remote_copy(
                src_ref=pbuf.at[rows((pos - 1) % N_DEV), CR],
                dst_ref=pbuf.at[rows((pos - 1) % N_DEV), CR],
                send_sem=send_ccw.at[2 + d], recv_sem=recv_ccw.at[2 + d],
                device_id=((pos + d) % N_DEV,),
                device_id_type=pl.DeviceIdType.MESH,
            )

        def conv(c, half):
            out_ref[rows(c), half] = pbuf[rows(c), half].astype(jnp.float32)

        q0 = jnp.dot(x_ref[rows(pos), :], wq_ref[:, :],
                     preferred_element_type=jnp.float32)
        kdma.wait()
        kbf[:, :] = kf32[:, :, :].reshape(SKV, H_PER * DH).astype(BF)
        vdma.wait()
        vbf[:, :] = vf32[:, :, :].reshape(SKV, H_PER * DH).astype(BF)

        compute_chunk(pos, q0)
        pl.semaphore_wait(barrier_sem, 2)
        cw_rdma(0).start()
        ccw_rdma(0).start()

        compute_chunk((pos + 1) % N_DEV)
        ccw_rdma(0).wait_recv()
        accum((pos + 1) % N_DEV, CR, stag_ccw, 0)
        ccw_rdma(1).start()

        compute_chunk((pos + 3) % N_DEV)
        cw_rdma(0).wait_recv()
        accum((pos - 1) % N_DEV, CL, stag_cw, 0)
        cw_rdma(1).start()

        compute_chunk((pos + 2) % N_DEV)
        ccw_rdma(1).wait_recv()
        accum((pos + 2) % N_DEV, CR, stag_ccw, 1)
        ccw_rdma(2).start()

        cw_rdma(1).wait_recv()
        accum((pos - 2) % N_DEV, CL, stag_cw, 1)
        cw_rdma(2).start()

        ccw_rdma(2).wait_recv()
        accum((pos + 3) % N_DEV, CR, stag_ccw, 2)
        ag_ccw(1).start()
        ag_ccw(2).start()
        ag_ccw(3).start()
        cw_rdma(2).wait_recv()
        accum((pos - 3) % N_DEV, CL, stag_cw, 2)
        ag_cw(1).start()
        ag_cw(2).start()
        ag_cw(3).start()

        conv((pos + 1) % N_DEV, CL)
        conv((pos - 1) % N_DEV, CR)

        for d in (1, 3, 2):
            ag_cw(d).wait_recv()
            conv((pos - d + 1) % N_DEV, CL)
            ag_ccw(d).wait_recv()
            conv((pos - d - 1) % N_DEV, CR)

        for s in range(3):
            cw_rdma(s).wait_send()
            ccw_rdma(s).wait_send()
        for d in (1, 2, 3):
            ag_cw(d).wait_send()
            ag_ccw(d).wait_send()

    out = pl.pallas_call(
        body,
        out_shape=jax.ShapeDtypeStruct((SQ, D_MODEL), jnp.float32),
        in_specs=[
            pl.BlockSpec(memory_space=pltpu.VMEM),
            pl.BlockSpec(memory_space=pltpu.VMEM),
            pl.BlockSpec(memory_space=pltpu.MemorySpace.HBM),
            pl.BlockSpec(memory_space=pltpu.MemorySpace.HBM),
            pl.BlockSpec(memory_space=pltpu.VMEM),
        ],
        out_specs=pl.BlockSpec(memory_space=pltpu.VMEM),
        scratch_shapes=[
            pltpu.VMEM((SQ // N_DEV, H_PER * DH), BF),
            pltpu.VMEM((SQ, D_MODEL), BF),
            pltpu.VMEM((3, SQ // N_DEV, D_MODEL // 2), BF),
            pltpu.VMEM((3, SQ // N_DEV, D_MODEL // 2), BF),
            pltpu.VMEM((SKV, H_PER, DH), jnp.float32),
            pltpu.VMEM((SKV, H_PER, DH), jnp.float32),
            pltpu.VMEM((SKV, H_PER * DH), BF),
            pltpu.VMEM((SKV, H_PER * DH), BF),
            pltpu.SemaphoreType.DMA((6,)),
            pltpu.SemaphoreType.DMA((6,)),
            pltpu.SemaphoreType.DMA((6,)),
            pltpu.SemaphoreType.DMA((6,)),
            pltpu.SemaphoreType.DMA((2,)),
        ],
        compiler_params=pltpu.CompilerParams(collective_id=0),
    )(x2, wq2, K_ext, V_ext, wo2)

    return out.reshape(1, SQ, D_MODEL)
